# baseline (speedup 1.0000x reference)
import jax
import jax.numpy as jnp
from jax.experimental import pallas as pl
from jax.experimental.pallas import tpu as pltpu

N = 10000
F = 128
NOUT = 64
BIG_B = 400
PRE_B = 1000


def _dot(a, b):
    return jax.lax.dot_general(a, b, (((1,), (0,)), ((), ())),
                               preferred_element_type=jnp.float32)


def _pre1_kernel(x_ref, wn_ref, ws_ref, b_ref, s_ref, h_ref):
    x = x_ref[...]
    s_ref[...] = _dot(x, wn_ref[...])
    h_ref[...] = _dot(x, ws_ref[...]) + b_ref[...]


def _pass1_kernel(adj_ref, s1_ref, h1_ref, g1_ref, g2_ref, g1b_ref, g2b_ref,
                  x_ref, l1_ref, l2_ref):
    x = jnp.maximum(_dot(adj_ref[...], s1_ref[...]) + h1_ref[...], 0.0)
    x_ref[...] = x
    l1_ref[...] = jnp.sum(x * g1_ref[...], axis=1, keepdims=True) + g1b_ref[0, 0]
    l2_ref[...] = jnp.sum(x * g2_ref[...], axis=1, keepdims=True) + g2b_ref[0, 0]


def _gates_kernel(l1_ref, l2_ref, r_ref, z_ref):
    l1 = l1_ref[...]
    e1 = jnp.exp(l1 - jnp.max(l1))
    r_ref[...] = e1 / jnp.sum(e1)
    l2 = l2_ref[...]
    e2 = jnp.exp(l2 - jnp.max(l2))
    z_ref[...] = e2 / jnp.sum(e2)


def _pre2_kernel(x_ref, r_ref, wn_ref, ws_ref, b_ref, s_ref, h_ref):
    xr = x_ref[...] * r_ref[...]
    s_ref[...] = _dot(xr, wn_ref[...])
    h_ref[...] = _dot(xr, ws_ref[...]) + b_ref[...]


def _pass2_kernel(adj_ref, s2_ref, h2_ref, x_ref, z_ref, pw_ref, pb_ref,
                  zenc_ref, pred_ref, acc_ref):
    i = pl.program_id(0)
    x2 = jnp.maximum(_dot(adj_ref[...], s2_ref[...]) + h2_ref[...], 0.0)
    z = z_ref[...]
    zenc = (1.0 - z) * x_ref[...] + z * x2
    zenc_ref[...] = zenc

    @pl.when(i == 0)
    def _():
        acc_ref[...] = jnp.zeros_like(acc_ref)

    acc_ref[...] += jnp.sum(zenc, axis=0, keepdims=True)

    @pl.when(i == pl.num_programs(0) - 1)
    def _():
        pred_ref[...] = _dot(acc_ref[...] * (1.0 / N), pw_ref[...]) + pb_ref[...]


def kernel(inputs, adj, Wn1, Ws1, b1, Wn2, Ws2, b2, g1w, g1b, g2w, g2b,
           e2pw, e2pb):
    f32 = jnp.float32
    b1r = b1.reshape(1, F)
    b2r = b2.reshape(1, F)
    g1t = g1w.reshape(1, F)
    g2t = g2w.reshape(1, F)
    g1br = g1b.reshape(1, 1)
    g2br = g2b.reshape(1, 1)
    pbr = e2pb.reshape(1, NOUT)

    full = lambda shape: pl.BlockSpec(shape, lambda i: (0,) * len(shape))
    rows = lambda nb, w: pl.BlockSpec((nb, w), lambda i: (i, 0))

    s1, h1 = pl.pallas_call(
        _pre1_kernel,
        grid=(N // PRE_B,),
        in_specs=[rows(PRE_B, F), full((F, F)), full((F, F)), full((1, F))],
        out_specs=[rows(PRE_B, F), rows(PRE_B, F)],
        out_shape=[jax.ShapeDtypeStruct((N, F), f32)] * 2,
    )(inputs, Wn1, Ws1, b1r)

    x, l1, l2 = pl.pallas_call(
        _pass1_kernel,
        grid=(N // BIG_B,),
        in_specs=[rows(BIG_B, N), full((N, F)), rows(BIG_B, F),
                  full((1, F)), full((1, F)), full((1, 1)), full((1, 1))],
        out_specs=[rows(BIG_B, F), rows(BIG_B, 1), rows(BIG_B, 1)],
        out_shape=[jax.ShapeDtypeStruct((N, F), f32),
                   jax.ShapeDtypeStruct((N, 1), f32),
                   jax.ShapeDtypeStruct((N, 1), f32)],
    )(adj, s1, h1, g1t, g2t, g1br, g2br)

    r, z = pl.pallas_call(
        _gates_kernel,
        grid=(1,),
        in_specs=[full((N, 1)), full((N, 1))],
        out_specs=[full((N, 1)), full((N, 1))],
        out_shape=[jax.ShapeDtypeStruct((N, 1), f32)] * 2,
    )(l1, l2)

    s2, h2 = pl.pallas_call(
        _pre2_kernel,
        grid=(N // PRE_B,),
        in_specs=[rows(PRE_B, F), rows(PRE_B, 1), full((F, F)), full((F, F)),
                  full((1, F))],
        out_specs=[rows(PRE_B, F), rows(PRE_B, F)],
        out_shape=[jax.ShapeDtypeStruct((N, F), f32)] * 2,
    )(x, r, Wn2, Ws2, b2r)

    zenc, pred = pl.pallas_call(
        _pass2_kernel,
        grid=(N // BIG_B,),
        in_specs=[rows(BIG_B, N), full((N, F)), rows(BIG_B, F),
                  rows(BIG_B, F), rows(BIG_B, 1), full((F, NOUT)),
                  full((1, NOUT))],
        out_specs=[rows(BIG_B, F), full((1, NOUT))],
        out_shape=[jax.ShapeDtypeStruct((N, F), f32),
                   jax.ShapeDtypeStruct((1, NOUT), f32)],
        scratch_shapes=[pltpu.VMEM((1, F), f32)],
    )(adj, s2, h2, x, z, e2pw, pbr)

    return (zenc, pred)


# speedup vs baseline: 1.1423x; 1.1423x over previous
import jax
import jax.numpy as jnp
from jax.experimental import pallas as pl
from jax.experimental.pallas import tpu as pltpu

N = 10000
F = 128
NOUT = 64
B1 = 400
B2 = 200


def _dot(a, b):
    return jax.lax.dot_general(a, b, (((1,), (0,)), ((), ())),
                               preferred_element_type=jnp.float32)


def _softmax_col(l):
    e = jnp.exp(l - jnp.max(l))
    return e / jnp.sum(e)


def _pass1_kernel(adj_ref, x_in_ref, wn_ref, ws_ref, b_ref, x_ref, s1_ref):
    i = pl.program_id(0)

    @pl.when(i == 0)
    def _():
        s1_ref[...] = _dot(x_in_ref[...], wn_ref[...])

    rows = x_in_ref[pl.ds(i * B1, B1), :]
    h1 = _dot(rows, ws_ref[...]) + b_ref[...]
    x_ref[...] = jnp.maximum(_dot(adj_ref[...], s1_ref[...]) + h1, 0.0)


def _pass2_kernel(adj_ref, x_ref, wn_ref, ws_ref, b_ref, g1_ref, g2_ref,
                  g1b_ref, g2b_ref, pw_ref, pb_ref,
                  zenc_ref, pred_ref, s2_ref, r_ref, z_ref, acc_ref):
    i = pl.program_id(0)

    @pl.when(i == 0)
    def _():
        x = x_ref[...]
        r_ref[...] = _softmax_col(_dot(x, g1_ref[...]) + g1b_ref[0, 0])
        z_ref[...] = _softmax_col(_dot(x, g2_ref[...]) + g2b_ref[0, 0])
        s2_ref[...] = _dot(x * r_ref[...], wn_ref[...])
        acc_ref[...] = jnp.zeros_like(acc_ref)

    x_blk = x_ref[pl.ds(i * B2, B2), :]
    r_blk = r_ref[pl.ds(i * B2, B2), :]
    z_blk = z_ref[pl.ds(i * B2, B2), :]
    h2 = _dot(x_blk * r_blk, ws_ref[...]) + b_ref[...]
    x2 = jnp.maximum(_dot(adj_ref[...], s2_ref[...]) + h2, 0.0)
    zenc = (1.0 - z_blk) * x_blk + z_blk * x2
    zenc_ref[...] = zenc
    acc_ref[...] += jnp.sum(zenc, axis=0, keepdims=True)

    @pl.when(i == pl.num_programs(0) - 1)
    def _():
        pred_ref[...] = _dot(acc_ref[...] * (1.0 / N), pw_ref[...]) + pb_ref[...]


def kernel(inputs, adj, Wn1, Ws1, b1, Wn2, Ws2, b2, g1w, g1b, g2w, g2b,
           e2pw, e2pb):
    f32 = jnp.float32
    full = lambda shape: pl.BlockSpec(shape, lambda i: (0,) * len(shape))
    rows = lambda nb, w: pl.BlockSpec((nb, w), lambda i: (i, 0))
    x = pl.pallas_call(
        _pass1_kernel,
        grid=(N // B1,),
        in_specs=[rows(B1, N), full((N, F)), full((F, F)), full((F, F)),
                  full((1, F))],
        out_specs=rows(B1, F),
        out_shape=jax.ShapeDtypeStruct((N, F), f32),
        scratch_shapes=[pltpu.VMEM((N, F), f32)],
    )(adj, inputs, Wn1, Ws1, b1.reshape(1, F))

    zenc, pred = pl.pallas_call(
        _pass2_kernel,
        grid=(N // B2,),
        in_specs=[rows(B2, N), full((N, F)), full((F, F)), full((F, F)),
                  full((1, F)), full((F, 1)), full((F, 1)), full((1, 1)),
                  full((1, 1)), full((F, NOUT)), full((1, NOUT))],
        out_specs=[rows(B2, F), full((1, NOUT))],
        out_shape=[jax.ShapeDtypeStruct((N, F), f32),
                   jax.ShapeDtypeStruct((1, NOUT), f32)],
        scratch_shapes=[pltpu.VMEM((N, F), f32), pltpu.VMEM((N, 1), f32),
                        pltpu.VMEM((N, 1), f32), pltpu.VMEM((1, F), f32)],
    )(adj, x, Wn2, Ws2, b2.reshape(1, F), g1w, g2w, g1b.reshape(1, 1),
      g2b.reshape(1, 1), e2pw, e2pb.reshape(1, NOUT))

    return (zenc, pred)


# speedup vs baseline: 1.1574x; 1.0132x over previous
import jax
import jax.numpy as jnp
from jax.experimental import pallas as pl
from jax.experimental.pallas import tpu as pltpu

N = 10000
F = 128
NOUT = 64
B = 200


def _dot(a, b):
    return jax.lax.dot_general(a, b, (((1,), (0,)), ((), ())),
                               preferred_element_type=jnp.float32)


def _softmax_col(l):
    e = jnp.exp(l - jnp.max(l))
    return e / jnp.sum(e)


def _fused_kernel(adj_ref, x_in_ref, wn1_ref, ws1_ref, b1_ref,
                  wn2_ref, ws2_ref, b2_ref, g1_ref, g2_ref, g1b_ref, g2b_ref,
                  pw_ref, pb_ref,
                  zenc_ref, pred_ref,
                  x_ref, s_ref, r_ref, z_ref, acc_ref):
    p = pl.program_id(0)
    i = pl.program_id(1)

    @pl.when((p == 0) & (i == 0))
    def _():
        s_ref[...] = _dot(x_in_ref[...], wn1_ref[...])

    @pl.when(p == 0)
    def _():
        rows = x_in_ref[pl.ds(i * B, B), :]
        h1 = _dot(rows, ws1_ref[...]) + b1_ref[...]
        x_ref[pl.ds(i * B, B), :] = jnp.maximum(
            _dot(adj_ref[...], s_ref[...]) + h1, 0.0)

    @pl.when((p == 1) & (i == 0))
    def _():
        x = x_ref[...]
        r_ref[...] = _softmax_col(_dot(x, g1_ref[...]) + g1b_ref[0, 0])
        z_ref[...] = _softmax_col(_dot(x, g2_ref[...]) + g2b_ref[0, 0])
        s_ref[...] = _dot(x * r_ref[...], wn2_ref[...])
        acc_ref[...] = jnp.zeros_like(acc_ref)

    @pl.when(p == 1)
    def _():
        x_blk = x_ref[pl.ds(i * B, B), :]
        r_blk = r_ref[pl.ds(i * B, B), :]
        z_blk = z_ref[pl.ds(i * B, B), :]
        h2 = _dot(x_blk * r_blk, ws2_ref[...]) + b2_ref[...]
        x2 = jnp.maximum(_dot(adj_ref[...], s_ref[...]) + h2, 0.0)
        zenc = (1.0 - z_blk) * x_blk + z_blk * x2
        zenc_ref[...] = zenc
        acc_ref[...] += jnp.sum(zenc, axis=0, keepdims=True)

    @pl.when((p == 1) & (i == pl.num_programs(1) - 1))
    def _():
        pred_ref[...] = _dot(acc_ref[...] * (1.0 / N), pw_ref[...]) + pb_ref[...]


def kernel(inputs, adj, Wn1, Ws1, b1, Wn2, Ws2, b2, g1w, g1b, g2w, g2b,
           e2pw, e2pb):
    f32 = jnp.float32
    full = lambda shape: pl.BlockSpec(shape, lambda p, i: (0,) * len(shape))

    zenc, pred = pl.pallas_call(
        _fused_kernel,
        grid=(2, N // B),
        in_specs=[pl.BlockSpec((B, N), lambda p, i: (i, 0)),
                  full((N, F)), full((F, F)), full((F, F)), full((1, F)),
                  full((F, F)), full((F, F)), full((1, F)),
                  full((F, 1)), full((F, 1)), full((1, 1)), full((1, 1)),
                  full((F, NOUT)), full((1, NOUT))],
        out_specs=[pl.BlockSpec((B, F), lambda p, i: (p * i, 0)),
                   full((1, NOUT))],
        out_shape=[jax.ShapeDtypeStruct((N, F), f32),
                   jax.ShapeDtypeStruct((1, NOUT), f32)],
        scratch_shapes=[pltpu.VMEM((N, F), f32), pltpu.VMEM((N, F), f32),
                        pltpu.VMEM((N, 1), f32), pltpu.VMEM((N, 1), f32),
                        pltpu.VMEM((1, F), f32)],
    )(adj, inputs, Wn1, Ws1, b1.reshape(1, F), Wn2, Ws2, b2.reshape(1, F),
      g1w, g2w, g1b.reshape(1, 1), g2b.reshape(1, 1), e2pw,
      e2pb.reshape(1, NOUT))

    return (zenc, pred)
